# baseline (speedup 1.0000x reference)
import functools

import jax
import jax.numpy as jnp
from jax import lax
from jax.experimental import pallas as pl
from jax.experimental.pallas import tpu as pltpu
from jax.experimental.pallas import tpu_sc as plsc

DIM = 64
BATCH = 16384
NC = 2
NS = 16
NW = NC * NS
BPW = BATCH // NW
CHUNK = 128
NCHUNK = BPW // CHUNK
NGROUP = BPW // 16


def _transe_body(bh, bt, br, ent, rel, out_hbm,
                 idx_h, idx_t, idx_r, hv, tv, rv, ov, sem):
    wid = lax.axis_index("s") * NC + lax.axis_index("c")
    base = wid * BPW

    pltpu.sync_copy(bh.at[pl.ds(wid * NCHUNK, NCHUNK)], idx_h)
    pltpu.sync_copy(bt.at[pl.ds(wid * NCHUNK, NCHUNK)], idx_t)
    pltpu.sync_copy(br.at[pl.ds(wid * NCHUNK, NCHUNK)], idx_r)

    copies = []
    for c in range(NCHUNK):
        copies.append(
            pltpu.async_copy(ent.at[idx_h.at[c]], hv.at[pl.ds(c * CHUNK, CHUNK)], sem))
        copies.append(
            pltpu.async_copy(ent.at[idx_t.at[c]], tv.at[pl.ds(c * CHUNK, CHUNK)], sem))
        copies.append(
            pltpu.async_copy(rel.at[idx_r.at[c]], rv.at[pl.ds(c * CHUNK, CHUNK)], sem))
    for cp in copies:
        cp.wait()

    lanes = lax.iota(jnp.int32, 16)
    perms = [lanes ^ (1 << b) for b in range(4)]

    dn = lax.GatherDimensionNumbers(
        offset_dims=(), collapsed_slice_dims=(0,), start_index_map=(0,))

    def shuffle(x, idx):
        return lax.gather(x, idx[:, None], dn, (1,),
                          mode=lax.GatherScatterMode.PROMISE_IN_BOUNDS)

    def lane_sum(s):
        for p in perms:
            s = s + shuffle(s, p)
        return s

    def group_body(g, _):
        acc = jnp.zeros((16,), jnp.float32)
        for k in range(16):
            i = g * 16 + k
            s = None
            for c in range(DIM // 16):
                a = jnp.abs(hv[i, pl.ds(c * 16, 16)]
                            + rv[i, pl.ds(c * 16, 16)]
                            - tv[i, pl.ds(c * 16, 16)])
                s = a if s is None else s + a
            acc = jnp.where(lanes == k, lane_sum(s), acc)
        ov[pl.ds(g * 16, 16)] = acc
        return 0

    lax.fori_loop(0, NGROUP, group_body, 0)

    pltpu.sync_copy(ov, out_hbm.at[pl.ds(base, BPW)])


_transe = functools.partial(
    pl.kernel,
    out_type=jax.ShapeDtypeStruct((BATCH,), jnp.float32),
    mesh=plsc.VectorSubcoreMesh(core_axis_name="c", subcore_axis_name="s"),
    scratch_types=[
        pltpu.VMEM((NCHUNK, CHUNK), jnp.int32),
        pltpu.VMEM((NCHUNK, CHUNK), jnp.int32),
        pltpu.VMEM((NCHUNK, CHUNK), jnp.int32),
        pltpu.VMEM((BPW, DIM), jnp.float32),
        pltpu.VMEM((BPW, DIM), jnp.float32),
        pltpu.VMEM((BPW, DIM), jnp.float32),
        pltpu.VMEM((BPW,), jnp.float32),
        pltpu.SemaphoreType.DMA,
    ],
    compiler_params=pltpu.CompilerParams(use_tc_tiling_on_sc=False),
)(_transe_body)


@jax.jit
def kernel(batch_h, batch_t, batch_r, ent_emb, rel_emb):
    bh = batch_h.reshape(NW * NCHUNK, CHUNK)
    bt = batch_t.reshape(NW * NCHUNK, CHUNK)
    br = batch_r.reshape(NW * NCHUNK, CHUNK)
    return _transe(bh, bt, br, ent_emb, rel_emb)


# speedup vs baseline: 1.6517x; 1.6517x over previous
import functools

import jax
import jax.numpy as jnp
from jax import lax
from jax.experimental import pallas as pl
from jax.experimental.pallas import tpu as pltpu
from jax.experimental.pallas import tpu_sc as plsc

DIM = 64
BATCH = 16384
NC = 2
NS = 16
NW = NC * NS
BPW = BATCH // NW
C = 128
NCH = BPW // C


def _transe_body(bh, bt, br, ent, rel, out_hbm,
                 idx_h, idx_t, idx_r, hv, tv, rv, ov, sem):
    wid = lax.axis_index("s") * NC + lax.axis_index("c")
    base = wid * BPW

    pltpu.sync_copy(bh.at[pl.ds(base, BPW)], idx_h)
    pltpu.sync_copy(bt.at[pl.ds(base, BPW)], idx_t)
    pltpu.sync_copy(br.at[pl.ds(base, BPW)], idx_r)

    lanes = lax.iota(jnp.int32, 16)
    perms = [lanes ^ (1 << b) for b in range(4)]
    dn = lax.GatherDimensionNumbers(
        offset_dims=(), collapsed_slice_dims=(0,), start_index_map=(0,))

    def lane_sum(s):
        for p in perms:
            s = s + lax.gather(s, p[:, None], dn, (1,),
                               mode=lax.GatherScatterMode.PROMISE_IN_BOUNDS)
        return s

    for ch in range(NCH):
        @pl.loop(0, C // 16)
        def _fire(g):
            jh = idx_h[pl.ds(ch * C + g * 16, 16)]
            jt = idx_t[pl.ds(ch * C + g * 16, 16)]
            jr = idx_r[pl.ds(ch * C + g * 16, 16)]
            for k in range(16):
                pltpu.async_copy(ent.at[pl.ds(jh[k], 1)],
                                 hv.at[pl.ds(g * 16 + k, 1)], sem)
                pltpu.async_copy(ent.at[pl.ds(jt[k], 1)],
                                 tv.at[pl.ds(g * 16 + k, 1)], sem)
                pltpu.async_copy(rel.at[pl.ds(jr[k], 1)],
                                 rv.at[pl.ds(g * 16 + k, 1)], sem)

        @pl.loop(0, C, unroll=8)
        def _drain(i):
            pltpu.make_async_copy(ent.at[pl.ds(0, 1)],
                                  hv.at[pl.ds(i, 1)], sem).wait()
            pltpu.make_async_copy(ent.at[pl.ds(0, 1)],
                                  tv.at[pl.ds(i, 1)], sem).wait()
            pltpu.make_async_copy(rel.at[pl.ds(0, 1)],
                                  rv.at[pl.ds(i, 1)], sem).wait()

        def group_body(g, _):
            acc = jnp.zeros((16,), jnp.float32)
            for k in range(16):
                i = g * 16 + k
                s = None
                for c in range(DIM // 16):
                    a = jnp.abs(hv[i, pl.ds(c * 16, 16)]
                                + rv[i, pl.ds(c * 16, 16)]
                                - tv[i, pl.ds(c * 16, 16)])
                    s = a if s is None else s + a
                acc = jnp.where(lanes == k, lane_sum(s), acc)
            ov[pl.ds(ch * C + g * 16, 16)] = acc
            return 0

        lax.fori_loop(0, C // 16, group_body, 0)

    pltpu.sync_copy(ov, out_hbm.at[pl.ds(base, BPW)])


_transe = functools.partial(
    pl.kernel,
    out_type=jax.ShapeDtypeStruct((BATCH,), jnp.float32),
    mesh=plsc.VectorSubcoreMesh(core_axis_name="c", subcore_axis_name="s"),
    scratch_types=[
        pltpu.VMEM((BPW,), jnp.int32),
        pltpu.VMEM((BPW,), jnp.int32),
        pltpu.VMEM((BPW,), jnp.int32),
        pltpu.VMEM((C, DIM), jnp.float32),
        pltpu.VMEM((C, DIM), jnp.float32),
        pltpu.VMEM((C, DIM), jnp.float32),
        pltpu.VMEM((BPW,), jnp.float32),
        pltpu.SemaphoreType.DMA,
    ],
)(_transe_body)


@jax.jit
def kernel(batch_h, batch_t, batch_r, ent_emb, rel_emb):
    return _transe(batch_h, batch_t, batch_r, ent_emb, rel_emb)
